# baseline (speedup 1.0000x reference)
import functools

import jax
import jax.numpy as jnp
from jax import lax
from jax.experimental import pallas as pl
from jax.experimental.pallas import tpu as pltpu
from jax.experimental.pallas import tpu_sc as plsc

N_NODES_ = 10000
N_EDGES_ = 320000
D_ = 128

NC = 2
NS = 16
NW = NC * NS
E_PER_TILE = N_EDGES_ // NW
CHUNK = 80
N_CHUNKS = 128
E_PAD_TILE = N_CHUNKS * CHUNK
N_PAD = E_PAD_TILE - E_PER_TILE
PHASES = (32, 32, 32, 32)
PC_MAX = 32
ACC_ROWS = N_NODES_ + N_PAD
ROWS_PER_TILE = 624
TAIL_BASE = NS * ROWS_PER_TILE


def _sc_body(src_hbm, dst_hbm, emb_hbm, zeros_hbm, out_hbm,
             srcA, srcB, dstA, dstB, rows0, rows1, rows2, acc,
             semr0, semr1, semr2, sems0, sems1, sems2, semsi, semdi):
  c = lax.axis_index("c")
  s = lax.axis_index("s")
  t = c * NS + s
  row_base = s * ROWS_PER_TILE

  pltpu.sync_copy(zeros_hbm.at[pl.ds(row_base, ROWS_PER_TILE)],
                  acc.at[pl.ds(row_base, ROWS_PER_TILE)])

  @pl.when(s == 0)
  def _init_tail():
    pltpu.sync_copy(zeros_hbm.at[pl.ds(TAIL_BASE, ACC_ROWS - TAIL_BASE)],
                    acc.at[pl.ds(TAIL_BASE, ACC_ROWS - TAIL_BASE)])

  rows = (rows0, rows1, rows2)
  semr = (semr0, semr1, semr2)
  sems = (sems0, sems1, sems2)
  srcs = (srcA, srcB)
  dsts = (dstA, dstB)

  def stage(p, sync):
    base = sum(PHASES[:p])
    pc = PHASES[p]
    sv, dv = srcs[p % 2], dsts[p % 2]
    src_slice = src_hbm.at[t, pl.ds(base * CHUNK, pc * CHUNK)]
    dst_slice = dst_hbm.at[t, pl.ds(base, pc)]
    if sync:
      pltpu.sync_copy(src_slice, sv.at[pl.ds(0, pc * CHUNK)])
      pltpu.sync_copy(dst_slice, dv.at[pl.ds(0, pc)])
    else:
      pltpu.async_copy(src_slice, sv.at[pl.ds(0, pc * CHUNK)], semsi)
      pltpu.async_copy(dst_slice, dv.at[pl.ds(0, pc)], semdi)

  def stage_wait(p):
    base = sum(PHASES[:p])
    pc = PHASES[p]
    sv, dv = srcs[p % 2], dsts[p % 2]
    pltpu.make_async_copy(src_hbm.at[t, pl.ds(base * CHUNK, pc * CHUNK)],
                          sv.at[pl.ds(0, pc * CHUNK)], semsi).wait()
    pltpu.make_async_copy(dst_hbm.at[t, pl.ds(base, pc)],
                          dv.at[pl.ds(0, pc)], semdi).wait()

  stage(0, sync=True)
  plsc.subcore_barrier()

  def run_phase(p):
    pc = PHASES[p]
    sv, dv = srcs[p % 2], dsts[p % 2]

    def gather(j, b):
      pltpu.async_copy(emb_hbm.at[sv.at[pl.ds(j * CHUNK, CHUNK)]],
                       rows[b], semr[b])

    def wait_gather(b):
      pltpu.make_async_copy(emb_hbm.at[sv.at[pl.ds(0, CHUNK)]],
                            rows[b], semr[b]).wait()

    def wait_scatter(b):
      pltpu.make_async_copy(rows[b], acc.at[dv.at[0]], sems[b]).wait()

    def step(j, b, wait_sc, issue_g):
      nb = (b + 1) % 3
      wait_gather(b)
      if wait_sc:
        wait_scatter(nb)
      if issue_g:
        gather(j + 1, nb)
      pltpu.async_copy(rows[b], acc.at[dv.at[j]], sems[b], add=True)

    gather(0, 0)
    gather(1, 1)
    if p + 1 < len(PHASES):
      stage(p + 1, sync=False)

    step(0, 0, False, True)
    step(1, 1, False, True)

    def triple(i, carry):
      j = 3 * i + 2
      step(j, 2, True, True)
      step(j + 1, 0, True, True)
      step(j + 2, 1, True, True)
      return carry

    n_triples = (pc - 4) // 3
    lax.fori_loop(0, n_triples, triple, 0)
    for j in range(2 + 3 * n_triples, pc):
      step(j, j % 3, True, j + 1 < pc)

    wait_scatter((pc - 2) % 3)
    wait_scatter((pc - 1) % 3)
    if p + 1 < len(PHASES):
      stage_wait(p + 1)

  for p in range(len(PHASES)):
    run_phase(p)

  plsc.subcore_barrier()
  pltpu.sync_copy(acc.at[pl.ds(row_base, ROWS_PER_TILE)],
                  out_hbm.at[c, pl.ds(row_base, ROWS_PER_TILE)])

  @pl.when(s == 0)
  def _write_tail():
    pltpu.sync_copy(acc.at[pl.ds(TAIL_BASE, N_NODES_ - TAIL_BASE)],
                    out_hbm.at[c, pl.ds(TAIL_BASE, N_NODES_ - TAIL_BASE)])


@functools.partial(
    pl.kernel,
    out_type=jax.ShapeDtypeStruct((NC, N_NODES_, D_), jnp.float32),
    mesh=plsc.VectorSubcoreMesh(core_axis_name="c", subcore_axis_name="s"),
    scratch_types=[
        pltpu.VMEM((PC_MAX * CHUNK,), jnp.int32),
        pltpu.VMEM((PC_MAX * CHUNK,), jnp.int32),
        pltpu.VMEM((PC_MAX, CHUNK), jnp.int32),
        pltpu.VMEM((PC_MAX, CHUNK), jnp.int32),
        pltpu.VMEM((CHUNK, D_), jnp.float32),
        pltpu.VMEM((CHUNK, D_), jnp.float32),
        pltpu.VMEM((CHUNK, D_), jnp.float32),
        pltpu.VMEM_SHARED((ACC_ROWS, D_), jnp.float32),
        pltpu.SemaphoreType.DMA,
        pltpu.SemaphoreType.DMA,
        pltpu.SemaphoreType.DMA,
        pltpu.SemaphoreType.DMA,
        pltpu.SemaphoreType.DMA,
        pltpu.SemaphoreType.DMA,
        pltpu.SemaphoreType.DMA,
        pltpu.SemaphoreType.DMA,
    ],
)
def _sc_aggregate(src_hbm, dst_hbm, emb_hbm, zeros_hbm, out_hbm,
                  srcA, srcB, dstA, dstB, rows0, rows1, rows2, acc,
                  semr0, semr1, semr2, sems0, sems1, sems2, semsi, semdi):
  _sc_body(src_hbm, dst_hbm, emb_hbm, zeros_hbm, out_hbm,
           srcA, srcB, dstA, dstB, rows0, rows1, rows2, acc,
           semr0, semr1, semr2, sems0, sems1, sems2, semsi, semdi)


def _add_body(a_ref, b_ref, o_ref):
  o_ref[...] = a_ref[...] + b_ref[...]


def _combine(p0, p1):
  blk = 1000
  return pl.pallas_call(
      _add_body,
      out_shape=jax.ShapeDtypeStruct((N_NODES_, D_), jnp.float32),
      grid=(N_NODES_ // blk,),
      in_specs=[pl.BlockSpec((blk, D_), lambda i: (i, 0)),
                pl.BlockSpec((blk, D_), lambda i: (i, 0))],
      out_specs=pl.BlockSpec((blk, D_), lambda i: (i, 0)),
  )(p0, p1)


def kernel(mode, edge_index, entity_embed):
  del mode
  src = edge_index[0].reshape(NW, E_PER_TILE)
  dst = edge_index[1].reshape(NW, E_PER_TILE)
  src_p = jnp.pad(src, ((0, 0), (0, N_PAD)))
  trash = jnp.broadcast_to(N_NODES_ + jnp.arange(N_PAD, dtype=jnp.int32),
                           (NW, N_PAD))
  dst_p = jnp.concatenate([dst, trash], axis=1).reshape(NW, N_CHUNKS, CHUNK)
  zeros = jnp.zeros((ACC_ROWS, D_), jnp.float32)
  partials = _sc_aggregate(src_p, dst_p, entity_embed, zeros)
  return _combine(partials[0], partials[1])


# speedup vs baseline: 1.0688x; 1.0688x over previous
import functools

import jax
import jax.numpy as jnp
from jax import lax
from jax.experimental import pallas as pl
from jax.experimental.pallas import tpu as pltpu
from jax.experimental.pallas import tpu_sc as plsc

N_NODES_ = 10000
N_EDGES_ = 320000
D_ = 128

NC = 2
NS = 16
NW = NC * NS
E_PER_TILE = N_EDGES_ // NW
CHUNK = 80
N_CHUNKS = 128
E_PAD_TILE = N_CHUNKS * CHUNK
N_PAD = E_PAD_TILE - E_PER_TILE
PHASES = (32, 32, 32, 32)
PC_MAX = 32
ACC_ROWS = N_NODES_ + N_PAD
ROWS_PER_TILE = 624
TAIL_BASE = NS * ROWS_PER_TILE


def _sc_body(src_hbm, dst_hbm, emb_hbm, zeros_hbm, out_hbm,
             srcA, srcB, dstA, dstB, rows0, rows1, rows2, acc,
             semr0, semr1, semr2, sems0, sems1, sems2, semsi, semdi):
  c = lax.axis_index("c")
  s = lax.axis_index("s")
  t = c * NS + s
  row_base = s * ROWS_PER_TILE

  pltpu.sync_copy(zeros_hbm.at[pl.ds(row_base, ROWS_PER_TILE)],
                  acc.at[pl.ds(row_base, ROWS_PER_TILE)])

  @pl.when(s == 0)
  def _init_tail():
    pltpu.sync_copy(zeros_hbm.at[pl.ds(TAIL_BASE, ACC_ROWS - TAIL_BASE)],
                    acc.at[pl.ds(TAIL_BASE, ACC_ROWS - TAIL_BASE)])

  rows = (rows0, rows1, rows2)
  semr = (semr0, semr1, semr2)
  sems = (sems0, sems1, sems2)
  srcs = (srcA, srcB)
  dsts = (dstA, dstB)

  def stage(p, sync):
    base = sum(PHASES[:p])
    pc = PHASES[p]
    sv, dv = srcs[p % 2], dsts[p % 2]
    src_slice = src_hbm.at[t, pl.ds(base * CHUNK, pc * CHUNK)]
    dst_slice = dst_hbm.at[t, pl.ds(base, pc)]
    if sync:
      pltpu.sync_copy(src_slice, sv.at[pl.ds(0, pc * CHUNK)])
      pltpu.sync_copy(dst_slice, dv.at[pl.ds(0, pc)])
    else:
      pltpu.async_copy(src_slice, sv.at[pl.ds(0, pc * CHUNK)], semsi)
      pltpu.async_copy(dst_slice, dv.at[pl.ds(0, pc)], semdi)

  def stage_wait(p):
    base = sum(PHASES[:p])
    pc = PHASES[p]
    sv, dv = srcs[p % 2], dsts[p % 2]
    pltpu.make_async_copy(src_hbm.at[t, pl.ds(base * CHUNK, pc * CHUNK)],
                          sv.at[pl.ds(0, pc * CHUNK)], semsi).wait()
    pltpu.make_async_copy(dst_hbm.at[t, pl.ds(base, pc)],
                          dv.at[pl.ds(0, pc)], semdi).wait()

  stage(0, sync=True)
  plsc.subcore_barrier()

  def run_phase(p):
    pc = PHASES[p]
    sv, dv = srcs[p % 2], dsts[p % 2]

    def gather(j, b):
      pltpu.async_copy(emb_hbm.at[sv.at[pl.ds(j * CHUNK, CHUNK)]],
                       rows[b], semr[b])

    def wait_gather(b):
      pltpu.make_async_copy(emb_hbm.at[sv.at[pl.ds(0, CHUNK)]],
                            rows[b], semr[b]).wait()

    def wait_scatter(b):
      pltpu.make_async_copy(rows[b], acc.at[dv.at[0]], sems[b]).wait()

    def step(j, b, wait_sc, issue_g):
      nb = (b + 2) % 3
      wait_gather(b)
      pltpu.async_copy(rows[b], acc.at[dv.at[j]], sems[b], add=True)
      if wait_sc:
        wait_scatter(nb)
      if issue_g:
        gather(j + 2, nb)

    gather(0, 0)
    gather(1, 1)
    if p + 1 < len(PHASES):
      stage(p + 1, sync=False)

    step(0, 0, False, True)

    def triple(i, carry):
      j = 3 * i + 1
      step(j, 1, True, True)
      step(j + 1, 2, True, True)
      step(j + 2, 0, True, True)
      return carry

    n_triples = (pc - 3) // 3
    lax.fori_loop(0, n_triples, triple, 0)
    for j in range(1 + 3 * n_triples, pc):
      step(j, j % 3, True, j + 2 < pc)

    wait_scatter((pc - 1) % 3)
    if p + 1 < len(PHASES):
      stage_wait(p + 1)

  for p in range(len(PHASES)):
    run_phase(p)

  plsc.subcore_barrier()
  pltpu.sync_copy(acc.at[pl.ds(row_base, ROWS_PER_TILE)],
                  out_hbm.at[c, pl.ds(row_base, ROWS_PER_TILE)])

  @pl.when(s == 0)
  def _write_tail():
    pltpu.sync_copy(acc.at[pl.ds(TAIL_BASE, N_NODES_ - TAIL_BASE)],
                    out_hbm.at[c, pl.ds(TAIL_BASE, N_NODES_ - TAIL_BASE)])


@functools.partial(
    pl.kernel,
    out_type=jax.ShapeDtypeStruct((NC, N_NODES_, D_), jnp.float32),
    mesh=plsc.VectorSubcoreMesh(core_axis_name="c", subcore_axis_name="s"),
    scratch_types=[
        pltpu.VMEM((PC_MAX * CHUNK,), jnp.int32),
        pltpu.VMEM((PC_MAX * CHUNK,), jnp.int32),
        pltpu.VMEM((PC_MAX, CHUNK), jnp.int32),
        pltpu.VMEM((PC_MAX, CHUNK), jnp.int32),
        pltpu.VMEM((CHUNK, D_), jnp.float32),
        pltpu.VMEM((CHUNK, D_), jnp.float32),
        pltpu.VMEM((CHUNK, D_), jnp.float32),
        pltpu.VMEM_SHARED((ACC_ROWS, D_), jnp.float32),
        pltpu.SemaphoreType.DMA,
        pltpu.SemaphoreType.DMA,
        pltpu.SemaphoreType.DMA,
        pltpu.SemaphoreType.DMA,
        pltpu.SemaphoreType.DMA,
        pltpu.SemaphoreType.DMA,
        pltpu.SemaphoreType.DMA,
        pltpu.SemaphoreType.DMA,
    ],
)
def _sc_aggregate(src_hbm, dst_hbm, emb_hbm, zeros_hbm, out_hbm,
                  srcA, srcB, dstA, dstB, rows0, rows1, rows2, acc,
                  semr0, semr1, semr2, sems0, sems1, sems2, semsi, semdi):
  _sc_body(src_hbm, dst_hbm, emb_hbm, zeros_hbm, out_hbm,
           srcA, srcB, dstA, dstB, rows0, rows1, rows2, acc,
           semr0, semr1, semr2, sems0, sems1, sems2, semsi, semdi)


def _add_body(a_ref, b_ref, o_ref):
  o_ref[...] = a_ref[...] + b_ref[...]


def _combine(p0, p1):
  blk = 1000
  return pl.pallas_call(
      _add_body,
      out_shape=jax.ShapeDtypeStruct((N_NODES_, D_), jnp.float32),
      grid=(N_NODES_ // blk,),
      in_specs=[pl.BlockSpec((blk, D_), lambda i: (i, 0)),
                pl.BlockSpec((blk, D_), lambda i: (i, 0))],
      out_specs=pl.BlockSpec((blk, D_), lambda i: (i, 0)),
  )(p0, p1)


def kernel(mode, edge_index, entity_embed):
  del mode
  src = edge_index[0].reshape(NW, E_PER_TILE)
  dst = edge_index[1].reshape(NW, E_PER_TILE)
  src_p = jnp.pad(src, ((0, 0), (0, N_PAD)))
  trash = jnp.broadcast_to(N_NODES_ + jnp.arange(N_PAD, dtype=jnp.int32),
                           (NW, N_PAD))
  dst_p = jnp.concatenate([dst, trash], axis=1).reshape(NW, N_CHUNKS, CHUNK)
  zeros = jnp.zeros((ACC_ROWS, D_), jnp.float32)
  partials = _sc_aggregate(src_p, dst_p, entity_embed, zeros)
  return _combine(partials[0], partials[1])


# speedup vs baseline: 2.2780x; 2.1314x over previous
import functools

import jax
import jax.numpy as jnp
from jax import lax
from jax.experimental import pallas as pl
from jax.experimental.pallas import tpu as pltpu
from jax.experimental.pallas import tpu_sc as plsc

N_NODES_ = 10000
N_EDGES_ = 320000
D_ = 128

NC = 2
NS = 16
NW = NC * NS
E_PER_TILE = N_EDGES_ // NW
CHUNK = 80
N_CHUNKS = E_PER_TILE // CHUNK
ACC_ROWS = N_NODES_
ROWS_PER_TILE = 624
TAIL_BASE = NS * ROWS_PER_TILE


def _sc_body(src_hbm, dst_hbm, emb_hbm, zeros_hbm, out_hbm,
             src_v, dst_v, rows0, rows1, acc, semr0, semr1, sems0, sems1):
  c = lax.axis_index("c")
  s = lax.axis_index("s")
  t = c * NS + s
  row_base = s * ROWS_PER_TILE

  pltpu.sync_copy(zeros_hbm.at[pl.ds(row_base, ROWS_PER_TILE)],
                  acc.at[pl.ds(row_base, ROWS_PER_TILE)])

  @pl.when(s == 0)
  def _init_tail():
    pltpu.sync_copy(zeros_hbm.at[pl.ds(TAIL_BASE, ACC_ROWS - TAIL_BASE)],
                    acc.at[pl.ds(TAIL_BASE, ACC_ROWS - TAIL_BASE)])

  pltpu.sync_copy(src_hbm.at[t], src_v)
  pltpu.sync_copy(dst_hbm.at[t], dst_v)
  plsc.subcore_barrier()

  rows = (rows0, rows1)
  semr = (semr0, semr1)
  sems = (sems0, sems1)
  src_flat = src_v

  def gather(j, b):
    return pltpu.async_copy(
        emb_hbm.at[src_flat.at[pl.ds(j * CHUNK, CHUNK)]], rows[b], semr[b],
        priority=1)

  def step(j, b, first, last):
    nb = 1 - b
    pltpu.make_async_copy(emb_hbm.at[src_flat.at[pl.ds(0, CHUNK)]],
                          rows[b], semr[b]).wait()
    if not first:
      pltpu.make_async_copy(rows[nb], acc.at[dst_v.at[j]], sems[nb]).wait()
    if not last:
      gather(j + 1, nb)
    pltpu.async_copy(rows[b], acc.at[dst_v.at[j]], sems[b], add=True)

  gather(0, 0)
  step(0, 0, True, False)

  def chunk_pair(i, carry):
    j = 2 * i + 1
    step(j, 1, False, False)
    step(j + 1, 0, False, False)
    return carry

  lax.fori_loop(0, (N_CHUNKS - 2) // 2, chunk_pair, 0)
  for j in range(N_CHUNKS - 1 - (N_CHUNKS % 2), N_CHUNKS):
    step(j, j % 2, False, j == N_CHUNKS - 1)
  b_last = (N_CHUNKS - 1) % 2
  pltpu.make_async_copy(rows[b_last], acc.at[dst_v.at[0]],
                        sems[b_last]).wait()

  plsc.subcore_barrier()
  pltpu.sync_copy(acc.at[pl.ds(row_base, ROWS_PER_TILE)],
                  out_hbm.at[c, pl.ds(row_base, ROWS_PER_TILE)])

  @pl.when(s == 0)
  def _write_tail():
    pltpu.sync_copy(acc.at[pl.ds(TAIL_BASE, N_NODES_ - TAIL_BASE)],
                    out_hbm.at[c, pl.ds(TAIL_BASE, N_NODES_ - TAIL_BASE)])


@functools.partial(
    pl.kernel,
    out_type=jax.ShapeDtypeStruct((NC, N_NODES_, D_), jnp.float32),
    mesh=plsc.VectorSubcoreMesh(core_axis_name="c", subcore_axis_name="s"),
    scratch_types=[
        pltpu.VMEM((N_CHUNKS * CHUNK,), jnp.int32),
        pltpu.VMEM((N_CHUNKS, CHUNK), jnp.int32),
        pltpu.VMEM((CHUNK, D_), jnp.float32),
        pltpu.VMEM((CHUNK, D_), jnp.float32),
        pltpu.VMEM_SHARED((ACC_ROWS, D_), jnp.float32),
        pltpu.SemaphoreType.DMA,
        pltpu.SemaphoreType.DMA,
        pltpu.SemaphoreType.DMA,
        pltpu.SemaphoreType.DMA,
    ],
)
def _sc_aggregate(src_hbm, dst_hbm, emb_hbm, zeros_hbm, out_hbm,
                  src_v, dst_v, rows0, rows1, acc,
                  semr0, semr1, sems0, sems1):
  _sc_body(src_hbm, dst_hbm, emb_hbm, zeros_hbm, out_hbm,
           src_v, dst_v, rows0, rows1, acc, semr0, semr1, sems0, sems1)


def _add_body(a_ref, b_ref, o_ref):
  o_ref[...] = a_ref[...] + b_ref[...]


def _combine(p0, p1):
  blk = 1000
  return pl.pallas_call(
      _add_body,
      out_shape=jax.ShapeDtypeStruct((N_NODES_, D_), jnp.float32),
      grid=(N_NODES_ // blk,),
      in_specs=[pl.BlockSpec((blk, D_), lambda i: (i, 0)),
                pl.BlockSpec((blk, D_), lambda i: (i, 0))],
      out_specs=pl.BlockSpec((blk, D_), lambda i: (i, 0)),
  )(p0, p1)


def kernel(mode, edge_index, entity_embed):
  del mode
  src_p = edge_index[0].reshape(NW, E_PER_TILE)
  dst_p = edge_index[1].reshape(NW, N_CHUNKS, CHUNK)
  zeros = jnp.zeros((ACC_ROWS, D_), jnp.float32)
  partials = _sc_aggregate(src_p, dst_p, entity_embed, zeros)
  return _combine(partials[0], partials[1])
